# baseline (speedup 1.0000x reference)
import jax
import jax.numpy as jnp
from jax.experimental import pallas as pl
from jax.experimental.pallas import tpu as pltpu

_BS = 512


def _mask_fill_block(mask_ref, t_ref, o_ref):
    m = mask_ref[...] != 0
    o_ref[...] = jnp.where(m, t_ref[...], jnp.float32(0.0))


def kernel(tensor, mask):
    B, S, D = tensor.shape
    m3 = mask.astype(jnp.int32).reshape(B, S, 1)
    grid = (B, S // _BS)
    return pl.pallas_call(
        _mask_fill_block,
        grid=grid,
        in_specs=[
            pl.BlockSpec((1, _BS, 1), lambda b, s: (b, s, 0)),
            pl.BlockSpec((1, _BS, D), lambda b, s: (b, s, 0)),
        ],
        out_specs=pl.BlockSpec((1, _BS, D), lambda b, s: (b, s, 0)),
        out_shape=jax.ShapeDtypeStruct((B, S, D), tensor.dtype),
        compiler_params=pltpu.CompilerParams(
            dimension_semantics=("parallel", "parallel"),
        ),
    )(m3, tensor)
